# baseline (speedup 1.0000x reference)
import functools

import jax
import jax.numpy as jnp
from jax import lax
from jax.experimental import pallas as pl
from jax.experimental.pallas import tpu as pltpu
from jax.experimental.pallas import tpu_sc as plsc

NC = 2
NS = 16
L = 16
NW = NC * NS
CH = 80
NB = 3


def _sc_edge_agg(n, d, e):
    ept = e // NW
    nch = ept // CH
    wb = 1000
    nwb = n // wb
    zr = 200
    nz = wb // zr

    mesh = plsc.VectorSubcoreMesh(core_axis_name="c", subcore_axis_name="s")

    @functools.partial(
        pl.kernel,
        out_type=(
            jax.ShapeDtypeStruct((NC, n, d), jnp.float32),
            jax.ShapeDtypeStruct((NC, n, L), jnp.float32),
        ),
        mesh=mesh,
        compiler_params=pltpu.CompilerParams(
            needs_layout_passes=False, use_tc_tiling_on_sc=False),
        scratch_types=[
            pltpu.VMEM((n,), jnp.float32),
            pltpu.VMEM((n,), jnp.float32),
            pltpu.VMEM((nch, CH), jnp.int32),
            pltpu.VMEM((nch, CH), jnp.int32),
            [pltpu.VMEM((CH, d), jnp.float32) for _ in range(NB)],
            [pltpu.VMEM((CH, L), jnp.float32) for _ in range(NB)],
            pltpu.VMEM((zr, d), jnp.float32),
            pltpu.VMEM((zr, L), jnp.float32),
            pltpu.VMEM_SHARED((n, d), jnp.float32),
            pltpu.VMEM_SHARED((n, L), jnp.float32),
            [pltpu.SemaphoreType.DMA for _ in range(NB)],
            [pltpu.SemaphoreType.DMA for _ in range(NB)],
        ],
    )
    def k(h_hbm, as_hbm, ad_hbm, src_hbm, dst_hbm, outp, denp,
          asv, adv, srcv, dstv, rows, wrows, zb, dzb, acc, dacc, gsem, ssem):
        cid = lax.axis_index("c")
        sid = lax.axis_index("s")
        wid = sid * NC + cid

        iota = lax.iota(jnp.int32, L)
        zi = jnp.zeros((L,), jnp.int32)
        zf = jnp.zeros((L,), jnp.float32)

        def zrow(r, _):
            for j in range(d // L):
                zb[r, pl.ds(j * L, L)] = zf
            dzb[r, :] = zf
            return 0
        lax.fori_loop(0, zr, zrow, 0)

        def zwrow(r, _):
            for b in range(NB):
                wrows[b][r, :] = zf
            return 0
        lax.fori_loop(0, CH, zwrow, 0)

        base = sid * wb

        @pl.when(sid < nwb)
        def _init():
            for i in range(nz):
                pltpu.sync_copy(zb, acc.at[pl.ds(base + i * zr, zr)])
                pltpu.sync_copy(dzb, dacc.at[pl.ds(base + i * zr, zr)])

        pltpu.sync_copy(as_hbm, asv)
        pltpu.sync_copy(ad_hbm, adv)
        pltpu.sync_copy(src_hbm.at[wid], srcv)
        pltpu.sync_copy(dst_hbm.at[wid], dstv)

        plsc.subcore_barrier()

        ridx = [iota + (j * L) for j in range(CH // L)]
        main = (nch // NB) * NB

        def attn_weights(c, b):
            ws = []
            for j in range(CH // L):
                si = srcv[c, pl.ds(j * L, L)]
                di = dstv[c, pl.ds(j * L, L)]
                ev = plsc.load_gather(asv, [si]) + plsc.load_gather(adv, [di])
                ev = jnp.where(ev >= 0, ev, 0.2 * ev)
                w = jnp.exp(ev)
                ws.append(w)
                plsc.store_scatter(wrows[b], [ridx[j], zi], w)
            return ws

        def scale_rows(b, ws):
            uf = 4

            def row(rr, _):
                for u in range(uf):
                    r = rr * uf + u
                    wv = wrows[b][r, :]
                    bc = jnp.full((L,), wv[0], jnp.float32)
                    for j in range(d // L):
                        sl = pl.ds(j * L, L)
                        rows[b][r, sl] = rows[b][r, sl] * bc
                return 0
            lax.fori_loop(0, CH // uf, row, 0)

        def drain_scatter(b):
            pltpu.make_async_copy(
                h_hbm.at[pl.ds(0, CH)], rows[b], ssem[b]).wait()
            pltpu.make_async_copy(
                denp.at[0, pl.ds(0, CH)], wrows[b], ssem[b]).wait()

        for b in range(NB - 1):
            pltpu.async_copy(h_hbm.at[srcv.at[b]], rows[b], gsem[b])

        def outer(g, _):
            for b in range(NB):
                c = NB * g + b
                pltpu.make_async_copy(
                    h_hbm.at[srcv.at[c]], rows[b], gsem[b]).wait()
                ws = attn_weights(c, b)

                bp = (b + NB - 1) % NB

                @pl.when(c + NB - 1 < main)
                def _refill():
                    @pl.when(c >= 1)
                    def _():
                        drain_scatter(bp)
                    pltpu.async_copy(
                        h_hbm.at[srcv.at[c + NB - 1]], rows[bp], gsem[bp])

                scale_rows(b, ws)

                pltpu.async_copy(rows[b], acc.at[dstv.at[c]], ssem[b],
                                 add=True)
                pltpu.async_copy(wrows[b], dacc.at[dstv.at[c]], ssem[b],
                                 add=True)
            return 0

        lax.fori_loop(0, nch // NB, outer, 0)

        for b in range(NB):
            drain_scatter(b)

        for c in range(main, nch):
            pltpu.async_copy(h_hbm.at[srcv.at[c]], rows[0], gsem[0]).wait()
            ws = attn_weights(c, 0)
            scale_rows(0, ws)
            pltpu.sync_copy(rows[0], acc.at[dstv.at[c]], add=True)
            pltpu.sync_copy(wrows[0], dacc.at[dstv.at[c]], add=True)

        plsc.subcore_barrier()

        @pl.when(sid < nwb)
        def _writeback():
            pltpu.sync_copy(acc.at[pl.ds(base, wb)],
                            outp.at[cid, pl.ds(base, wb)])
            pltpu.sync_copy(dacc.at[pl.ds(base, wb)],
                            denp.at[cid, pl.ds(base, wb)])

    return k


def _tc_first(n, d_in, d_out, bn=2000):
    def body(x_ref, w_ref, a_ref, h_ref, am_ref):
        h = jnp.dot(x_ref[...], w_ref[...], preferred_element_type=jnp.float32)
        h_ref[...] = h
        am_ref[...] = jnp.dot(h, a_ref[...], preferred_element_type=jnp.float32)

    return pl.pallas_call(
        body,
        grid=(n // bn,),
        in_specs=[
            pl.BlockSpec((bn, d_in), lambda i: (i, 0)),
            pl.BlockSpec((d_in, d_out), lambda i: (0, 0)),
            pl.BlockSpec((d_out, 128), lambda i: (0, 0)),
        ],
        out_specs=[
            pl.BlockSpec((bn, d_out), lambda i: (i, 0)),
            pl.BlockSpec((bn, 128), lambda i: (i, 0)),
        ],
        out_shape=[
            jax.ShapeDtypeStruct((n, d_out), jnp.float32),
            jax.ShapeDtypeStruct((n, 128), jnp.float32),
        ],
    )


def _tc_combine(n, dp, dn, bn=2000):
    def body(op_ref, dp_ref, h_ref, am_ref, b_ref, w_ref, a_ref, hn_ref, amn_ref):
        am = am_ref[...]
        s = am[:, 0:1] + am[:, 1:2]
        wself = jnp.exp(jnp.where(s >= 0, s, 0.2 * s))
        den = dp_ref[0, :, 0:1] + dp_ref[1, :, 0:1] + wself
        num = op_ref[0] + op_ref[1] + wself * h_ref[...]
        agg = num / (den + 1e-16)
        xn = jnp.maximum(agg + b_ref[...], 0.0)
        hn = jnp.dot(xn, w_ref[...], preferred_element_type=jnp.float32)
        hn_ref[...] = hn
        amn_ref[...] = jnp.dot(hn, a_ref[...], preferred_element_type=jnp.float32)

    return pl.pallas_call(
        body,
        grid=(n // bn,),
        in_specs=[
            pl.BlockSpec((2, bn, dp), lambda i: (0, i, 0)),
            pl.BlockSpec((2, bn, L), lambda i: (0, i, 0)),
            pl.BlockSpec((bn, dp), lambda i: (i, 0)),
            pl.BlockSpec((bn, 128), lambda i: (i, 0)),
            pl.BlockSpec((1, dp), lambda i: (0, 0)),
            pl.BlockSpec((dp, dn), lambda i: (0, 0)),
            pl.BlockSpec((dn, 128), lambda i: (0, 0)),
        ],
        out_specs=[
            pl.BlockSpec((bn, dn), lambda i: (i, 0)),
            pl.BlockSpec((bn, 128), lambda i: (i, 0)),
        ],
        out_shape=[
            jax.ShapeDtypeStruct((n, dn), jnp.float32),
            jax.ShapeDtypeStruct((n, 128), jnp.float32),
        ],
    )


def _tc_final(n, dh, dp, bn=2000):
    def body(opa_ref, opb_ref, dp_ref, h_ref, am_ref, b_ref, out_ref):
        am = am_ref[...]
        s = am[:, 0:1] + am[:, 1:2]
        wself = jnp.exp(jnp.where(s >= 0, s, 0.2 * s))
        den = dp_ref[0, :, 0:1] + dp_ref[1, :, 0:1] + wself
        halves = jnp.concatenate(
            [opa_ref[0] + opa_ref[1], opb_ref[0] + opb_ref[1]], axis=1)
        num = halves + wself * h_ref[...]
        out_ref[...] = num / (den + 1e-16) + b_ref[...]

    return pl.pallas_call(
        body,
        grid=(n // bn,),
        in_specs=[
            pl.BlockSpec((2, bn, dh), lambda i: (0, i, 0)),
            pl.BlockSpec((2, bn, dh), lambda i: (0, i, 0)),
            pl.BlockSpec((2, bn, L), lambda i: (0, i, 0)),
            pl.BlockSpec((bn, dp), lambda i: (i, 0)),
            pl.BlockSpec((bn, 128), lambda i: (i, 0)),
            pl.BlockSpec((1, dp), lambda i: (0, 0)),
        ],
        out_specs=pl.BlockSpec((bn, dp), lambda i: (i, 0)),
        out_shape=jax.ShapeDtypeStruct((n, dp), jnp.float32),
    )


def _pack_attn(a_s, a_d):
    d = a_s.shape[0]
    a = jnp.zeros((d, 128), jnp.float32)
    return a.at[:, 0].set(a_s).at[:, 1].set(a_d)


def kernel(x, edge_index, W0, a_s0, a_d0, b0, W1, a_s1, a_d1, b1,
           W2, a_s2, a_d2, b2):
    n, d_in = x.shape
    e = edge_index.shape[1]
    d_hid = W0.shape[1]
    d_out = W2.shape[1]

    src_r = edge_index[0].reshape(NW, (e // NW) // CH, CH)
    dst_r = edge_index[1].reshape(NW, (e // NW) // CH, CH)

    A0 = _pack_attn(a_s0, a_d0)
    A1 = _pack_attn(a_s1, a_d1)
    A2 = _pack_attn(a_s2, a_d2)

    sc_h = _sc_edge_agg(n, d_hid, e)

    h0, am0 = _tc_first(n, d_in, d_hid)(x, W0, A0)
    op0, dn0 = sc_h(h0, am0[:, 0], am0[:, 1], src_r, dst_r)
    h1, am1 = _tc_combine(n, d_hid, d_hid)(
        op0, dn0, h0, am0, b0.reshape(1, -1), W1, A1)
    op1, dn1 = sc_h(h1, am1[:, 0], am1[:, 1], src_r, dst_r)
    h2, am2 = _tc_combine(n, d_hid, d_out)(
        op1, dn1, h1, am1, b1.reshape(1, -1), W2, A2)
    op2a, dn2 = sc_h(h2[:, :d_hid], am2[:, 0], am2[:, 1], src_r, dst_r)
    op2b, _ = sc_h(h2[:, d_hid:], am2[:, 0], am2[:, 1], src_r, dst_r)
    out = _tc_final(n, d_hid, d_out)(
        op2a, op2b, dn2, h2, am2, b2.reshape(1, -1))
    return out


# speedup vs baseline: 1.0854x; 1.0854x over previous
import functools

import jax
import jax.numpy as jnp
from jax import lax
from jax.experimental import pallas as pl
from jax.experimental.pallas import tpu as pltpu
from jax.experimental.pallas import tpu_sc as plsc

NC = 2
NS = 16
L = 16
NW = NC * NS
CH = 80
NB = 3


def _sc_edge_agg(n, d, e):
    ept = e // NW
    nch = ept // CH
    wb = 1000
    nwb = n // wb
    zr = 200
    nz = wb // zr

    mesh = plsc.VectorSubcoreMesh(core_axis_name="c", subcore_axis_name="s")

    @functools.partial(
        pl.kernel,
        out_type=(
            jax.ShapeDtypeStruct((NC, n, d), jnp.float32),
            jax.ShapeDtypeStruct((NC, n, L), jnp.float32),
        ),
        mesh=mesh,
        compiler_params=pltpu.CompilerParams(
            needs_layout_passes=False, use_tc_tiling_on_sc=False),
        scratch_types=[
            pltpu.VMEM((n,), jnp.float32),
            pltpu.VMEM((n,), jnp.float32),
            pltpu.VMEM((nch, CH), jnp.int32),
            pltpu.VMEM((nch, CH), jnp.int32),
            [pltpu.VMEM((CH, d), jnp.float32) for _ in range(NB)],
            [pltpu.VMEM((CH, L), jnp.float32) for _ in range(NB)],
            pltpu.VMEM((zr, d), jnp.float32),
            pltpu.VMEM((zr, L), jnp.float32),
            pltpu.VMEM_SHARED((n, d), jnp.float32),
            pltpu.VMEM_SHARED((n, L), jnp.float32),
            [pltpu.SemaphoreType.DMA for _ in range(NB)],
            [pltpu.SemaphoreType.DMA for _ in range(NB)],
        ],
    )
    def k(h_hbm, as_hbm, ad_hbm, src_hbm, dst_hbm, outp, denp,
          asv, adv, srcv, dstv, rows, wrows, zb, dzb, acc, dacc, gsem, ssem):
        cid = lax.axis_index("c")
        sid = lax.axis_index("s")
        wid = sid * NC + cid

        iota = lax.iota(jnp.int32, L)
        zi = jnp.zeros((L,), jnp.int32)
        zf = jnp.zeros((L,), jnp.float32)

        def zrow(r, _):
            for j in range(d // L):
                zb[r, pl.ds(j * L, L)] = zf
            dzb[r, :] = zf
            return 0
        lax.fori_loop(0, zr, zrow, 0)

        def zwrow(r, _):
            for b in range(NB):
                wrows[b][r, :] = zf
            return 0
        lax.fori_loop(0, CH, zwrow, 0)

        base = sid * wb

        @pl.when(sid < nwb)
        def _init():
            for i in range(nz):
                pltpu.sync_copy(zb, acc.at[pl.ds(base + i * zr, zr)])
                pltpu.sync_copy(dzb, dacc.at[pl.ds(base + i * zr, zr)])

        pltpu.sync_copy(as_hbm, asv)
        pltpu.sync_copy(ad_hbm, adv)
        pltpu.sync_copy(src_hbm.at[wid], srcv)
        pltpu.sync_copy(dst_hbm.at[wid], dstv)

        plsc.subcore_barrier()

        ridx = [iota + (j * L) for j in range(CH // L)]
        main = (nch // NB) * NB

        def attn_weights(c, b):
            ws = []
            for j in range(CH // L):
                si = srcv[c, pl.ds(j * L, L)]
                di = dstv[c, pl.ds(j * L, L)]
                ev = plsc.load_gather(asv, [si]) + plsc.load_gather(adv, [di])
                ev = jnp.where(ev >= 0, ev, 0.2 * ev)
                w = jnp.exp(ev)
                ws.append(w)
                plsc.store_scatter(wrows[b], [ridx[j], zi], w)
            return ws

        def scale_rows(b, ws):
            uf = 2

            def row(rr, _):
                for u in range(uf):
                    r = rr * uf + u
                    wv = wrows[b][r, :]
                    bc = jnp.full((L,), wv[0], jnp.float32)
                    for j in range(d // L):
                        sl = pl.ds(j * L, L)
                        rows[b][r, sl] = rows[b][r, sl] * bc
                return 0
            lax.fori_loop(0, CH // uf, row, 0)

        def drain_scatter(b):
            pltpu.make_async_copy(
                h_hbm.at[pl.ds(0, CH)], rows[b], ssem[b]).wait()
            pltpu.make_async_copy(
                denp.at[0, pl.ds(0, CH)], wrows[b], ssem[b]).wait()

        for b in range(NB - 1):
            pltpu.async_copy(h_hbm.at[srcv.at[b]], rows[b], gsem[b])

        def outer(g, _):
            for b in range(NB):
                c = NB * g + b
                pltpu.make_async_copy(
                    h_hbm.at[srcv.at[c]], rows[b], gsem[b]).wait()
                ws = attn_weights(c, b)
                scale_rows(b, ws)

                bp = (b + NB - 1) % NB

                @pl.when(c + NB - 1 < main)
                def _refill():
                    @pl.when(c >= 1)
                    def _():
                        drain_scatter(bp)
                    pltpu.async_copy(
                        h_hbm.at[srcv.at[c + NB - 1]], rows[bp], gsem[bp])

                pltpu.async_copy(rows[b], acc.at[dstv.at[c]], ssem[b],
                                 add=True)
                pltpu.async_copy(wrows[b], dacc.at[dstv.at[c]], ssem[b],
                                 add=True)
            return 0

        lax.fori_loop(0, nch // NB, outer, 0)

        for b in range(NB):
            drain_scatter(b)

        for c in range(main, nch):
            pltpu.async_copy(h_hbm.at[srcv.at[c]], rows[0], gsem[0]).wait()
            ws = attn_weights(c, 0)
            scale_rows(0, ws)
            pltpu.sync_copy(rows[0], acc.at[dstv.at[c]], add=True)
            pltpu.sync_copy(wrows[0], dacc.at[dstv.at[c]], add=True)

        plsc.subcore_barrier()

        @pl.when(sid < nwb)
        def _writeback():
            pltpu.sync_copy(acc.at[pl.ds(base, wb)],
                            outp.at[cid, pl.ds(base, wb)])
            pltpu.sync_copy(dacc.at[pl.ds(base, wb)],
                            denp.at[cid, pl.ds(base, wb)])

    return k


def _tc_first(n, d_in, d_out, bn=2000):
    def body(x_ref, w_ref, a_ref, h_ref, am_ref):
        h = jnp.dot(x_ref[...], w_ref[...], preferred_element_type=jnp.float32)
        h_ref[...] = h
        am_ref[...] = jnp.dot(h, a_ref[...], preferred_element_type=jnp.float32)

    return pl.pallas_call(
        body,
        grid=(n // bn,),
        in_specs=[
            pl.BlockSpec((bn, d_in), lambda i: (i, 0)),
            pl.BlockSpec((d_in, d_out), lambda i: (0, 0)),
            pl.BlockSpec((d_out, 128), lambda i: (0, 0)),
        ],
        out_specs=[
            pl.BlockSpec((bn, d_out), lambda i: (i, 0)),
            pl.BlockSpec((bn, 128), lambda i: (i, 0)),
        ],
        out_shape=[
            jax.ShapeDtypeStruct((n, d_out), jnp.float32),
            jax.ShapeDtypeStruct((n, 128), jnp.float32),
        ],
    )


def _tc_combine(n, dp, dn, bn=2000):
    def body(op_ref, dp_ref, h_ref, am_ref, b_ref, w_ref, a_ref, hn_ref, amn_ref):
        am = am_ref[...]
        s = am[:, 0:1] + am[:, 1:2]
        wself = jnp.exp(jnp.where(s >= 0, s, 0.2 * s))
        den = dp_ref[0, :, 0:1] + dp_ref[1, :, 0:1] + wself
        num = op_ref[0] + op_ref[1] + wself * h_ref[...]
        agg = num / (den + 1e-16)
        xn = jnp.maximum(agg + b_ref[...], 0.0)
        hn = jnp.dot(xn, w_ref[...], preferred_element_type=jnp.float32)
        hn_ref[...] = hn
        amn_ref[...] = jnp.dot(hn, a_ref[...], preferred_element_type=jnp.float32)

    return pl.pallas_call(
        body,
        grid=(n // bn,),
        in_specs=[
            pl.BlockSpec((2, bn, dp), lambda i: (0, i, 0)),
            pl.BlockSpec((2, bn, L), lambda i: (0, i, 0)),
            pl.BlockSpec((bn, dp), lambda i: (i, 0)),
            pl.BlockSpec((bn, 128), lambda i: (i, 0)),
            pl.BlockSpec((1, dp), lambda i: (0, 0)),
            pl.BlockSpec((dp, dn), lambda i: (0, 0)),
            pl.BlockSpec((dn, 128), lambda i: (0, 0)),
        ],
        out_specs=[
            pl.BlockSpec((bn, dn), lambda i: (i, 0)),
            pl.BlockSpec((bn, 128), lambda i: (i, 0)),
        ],
        out_shape=[
            jax.ShapeDtypeStruct((n, dn), jnp.float32),
            jax.ShapeDtypeStruct((n, 128), jnp.float32),
        ],
    )


def _tc_final(n, dh, dp, bn=2000):
    def body(opa_ref, opb_ref, dp_ref, h_ref, am_ref, b_ref, out_ref):
        am = am_ref[...]
        s = am[:, 0:1] + am[:, 1:2]
        wself = jnp.exp(jnp.where(s >= 0, s, 0.2 * s))
        den = dp_ref[0, :, 0:1] + dp_ref[1, :, 0:1] + wself
        halves = jnp.concatenate(
            [opa_ref[0] + opa_ref[1], opb_ref[0] + opb_ref[1]], axis=1)
        num = halves + wself * h_ref[...]
        out_ref[...] = num / (den + 1e-16) + b_ref[...]

    return pl.pallas_call(
        body,
        grid=(n // bn,),
        in_specs=[
            pl.BlockSpec((2, bn, dh), lambda i: (0, i, 0)),
            pl.BlockSpec((2, bn, dh), lambda i: (0, i, 0)),
            pl.BlockSpec((2, bn, L), lambda i: (0, i, 0)),
            pl.BlockSpec((bn, dp), lambda i: (i, 0)),
            pl.BlockSpec((bn, 128), lambda i: (i, 0)),
            pl.BlockSpec((1, dp), lambda i: (0, 0)),
        ],
        out_specs=pl.BlockSpec((bn, dp), lambda i: (i, 0)),
        out_shape=jax.ShapeDtypeStruct((n, dp), jnp.float32),
    )


def _pack_attn(a_s, a_d):
    d = a_s.shape[0]
    a = jnp.zeros((d, 128), jnp.float32)
    return a.at[:, 0].set(a_s).at[:, 1].set(a_d)


def kernel(x, edge_index, W0, a_s0, a_d0, b0, W1, a_s1, a_d1, b1,
           W2, a_s2, a_d2, b2):
    n, d_in = x.shape
    e = edge_index.shape[1]
    d_hid = W0.shape[1]
    d_out = W2.shape[1]

    src_r = edge_index[0].reshape(NW, (e // NW) // CH, CH)
    dst_r = edge_index[1].reshape(NW, (e // NW) // CH, CH)

    A0 = _pack_attn(a_s0, a_d0)
    A1 = _pack_attn(a_s1, a_d1)
    A2 = _pack_attn(a_s2, a_d2)

    sc_h = _sc_edge_agg(n, d_hid, e)

    h0, am0 = _tc_first(n, d_in, d_hid)(x, W0, A0)
    op0, dn0 = sc_h(h0, am0[:, 0], am0[:, 1], src_r, dst_r)
    h1, am1 = _tc_combine(n, d_hid, d_hid)(
        op0, dn0, h0, am0, b0.reshape(1, -1), W1, A1)
    op1, dn1 = sc_h(h1, am1[:, 0], am1[:, 1], src_r, dst_r)
    h2, am2 = _tc_combine(n, d_hid, d_out)(
        op1, dn1, h1, am1, b1.reshape(1, -1), W2, A2)
    op2a, dn2 = sc_h(h2[:, :d_hid], am2[:, 0], am2[:, 1], src_r, dst_r)
    op2b, _ = sc_h(h2[:, d_hid:], am2[:, 0], am2[:, 1], src_r, dst_r)
    out = _tc_final(n, d_hid, d_out)(
        op2a, op2b, dn2, h2, am2, b2.reshape(1, -1))
    return out
